# baseline (speedup 1.0000x reference)
import functools

import jax
import jax.numpy as jnp
from jax import lax
from jax.experimental import pallas as pl
from jax.experimental.pallas import tpu as pltpu
from jax.experimental.pallas import tpu_sc as plsc

B = 4
S = 4096
D = 768
N_ROWS = B * S
NC = 2
NS = 16
NW = NC * NS
S_PER_W = S // NW
CHUNK = 32
N_PCH = S_PER_W // CHUNK
N_CHUNKS = N_PCH * B
LANES = 16
D2 = D // 32
SCALE = 27.712812921102035


def _sc_body(x_hbm, pe_hbm, table_hbm, out_hbm,
             idx_v, r0, r1, r2, pef, pepk,
             g0, g1, g2, p0, s0_, s1_, s2_):
    rows = [r0, r1, r2]
    gsem = [g0, g1, g2]
    ssem = [s0_, s1_, s2_]

    wid = lax.axis_index("s") * NC + lax.axis_index("c")
    w0 = wid * S_PER_W

    for b in range(B):
        pltpu.sync_copy(x_hbm.at[b, pl.ds(w0, S_PER_W)],
                        idx_v.at[pl.ds(b * S_PER_W, S_PER_W)])

    def flat_base(t):
        cc, b = t // B, t % B
        return b * S + w0 + cc * CHUNK

    def start_gather(t):
        cc, b = t // B, t % B
        off = b * S_PER_W + cc * CHUNK
        return pltpu.async_copy(
            table_hbm.at[idx_v.at[pl.ds(off, CHUNK)]],
            rows[t % 3], gsem[t % 3])

    def start_pe(cc):
        return pltpu.async_copy(
            pe_hbm.at[pl.ds(w0 + cc * CHUNK, CHUNK)], pef, p0)

    g_h = [None, None, None]
    s_h = [None, None, None]
    g_h[0] = start_gather(0)
    g_h[1] = start_gather(1)
    p_h = start_pe(0)

    for t in range(N_CHUNKS):
        cc, b = t // B, t % B
        rb = t % 3

        if b == 0:
            p_h.wait()

            def pack_body(r, _):
                for db in range(D2):
                    a = lax.bitcast_convert_type(
                        pef[r, pl.ds(32 * db, LANES)], jnp.uint32)
                    bb = lax.bitcast_convert_type(
                        pef[r, pl.ds(32 * db + LANES, LANES)], jnp.uint32)
                    pepk[r, pl.ds(LANES * db, LANES)] = (
                        (a >> jnp.uint32(16)) | (bb & jnp.uint32(0xFFFF0000)))
                return 0

            lax.fori_loop(0, CHUNK, pack_body, 0)
            if cc + 1 < N_PCH:
                p_h = start_pe(cc + 1)

        g_h[rb].wait()

        def row_body(r, _, _rb=rb):
            rr = rows[_rb]
            for db in range(D2):
                w = pepk[r, pl.ds(LANES * db, LANES)]
                pa = lax.bitcast_convert_type(w << jnp.uint32(16),
                                              jnp.float32)
                pb_ = lax.bitcast_convert_type(w & jnp.uint32(0xFFFF0000),
                                               jnp.float32)
                sla = pl.ds(32 * db, LANES)
                slb = pl.ds(32 * db + LANES, LANES)
                rr[r, sla] = rr[r, sla] * SCALE + pa
                rr[r, slb] = rr[r, slb] * SCALE + pb_
            return 0

        lax.fori_loop(0, CHUNK, row_body, 0)

        s_h[rb] = pltpu.async_copy(
            rows[rb], out_hbm.at[pl.ds(flat_base(t), CHUNK)], ssem[rb])

        nxt = t + 2
        if nxt < N_CHUNKS:
            if t >= 1:
                s_h[nxt % 3].wait()
            g_h[nxt % 3] = start_gather(nxt)

    for t in range(N_CHUNKS - 3, N_CHUNKS):
        s_h[t % 3].wait()


@jax.jit
def _embed(x, pe, table):
    mesh = plsc.VectorSubcoreMesh(core_axis_name="c", subcore_axis_name="s")
    k = functools.partial(
        pl.kernel,
        mesh=mesh,
        out_type=jax.ShapeDtypeStruct((N_ROWS, D), jnp.float32),
        scratch_types=[
            pltpu.VMEM((B * S_PER_W,), jnp.int32),
            pltpu.VMEM((CHUNK, D), jnp.float32),
            pltpu.VMEM((CHUNK, D), jnp.float32),
            pltpu.VMEM((CHUNK, D), jnp.float32),
            pltpu.VMEM((CHUNK, D), jnp.float32),
            pltpu.VMEM((CHUNK, D // 2), jnp.uint32),
            pltpu.SemaphoreType.DMA,
            pltpu.SemaphoreType.DMA,
            pltpu.SemaphoreType.DMA,
            pltpu.SemaphoreType.DMA,
            pltpu.SemaphoreType.DMA,
            pltpu.SemaphoreType.DMA,
            pltpu.SemaphoreType.DMA,
        ],
    )(_sc_body)
    return k(x, pe, table)


def kernel(x, token_table, pe):
    out = _embed(x.astype(jnp.int32), pe, token_table)
    return out.reshape(B, S, D)


# speedup vs baseline: 1.6636x; 1.6636x over previous
import functools

import jax
import jax.numpy as jnp
from jax import lax
from jax.experimental import pallas as pl
from jax.experimental.pallas import tpu as pltpu
from jax.experimental.pallas import tpu_sc as plsc

B = 4
S = 4096
D = 768
N_ROWS = B * S
NC = 2
NS = 16
NW = NC * NS
S_PER_W = S // NW
CHUNK = 32
N_PCH = S_PER_W // CHUNK
N_CHUNKS = N_PCH * B
LANES = 16
D_VECS = D // LANES
SCALE = 27.712812921102035


def _sc_body(x_hbm, pe_hbm, table_hbm, out_hbm,
             idx_v, r0, r1, r2, pv0, pv1,
             g0, g1, g2, p0, p1, s0_, s1_, s2_):
    rows = [r0, r1, r2]
    pes = [pv0, pv1]
    gsem = [g0, g1, g2]
    psem = [p0, p1]
    ssem = [s0_, s1_, s2_]

    wid = lax.axis_index("s") * NC + lax.axis_index("c")
    w0 = wid * S_PER_W

    for b in range(B):
        pltpu.sync_copy(x_hbm.at[b, pl.ds(w0, S_PER_W)],
                        idx_v.at[pl.ds(b * S_PER_W, S_PER_W)])

    def flat_base(t):
        cc, b = t // B, t % B
        return b * S + w0 + cc * CHUNK

    def start_gather(t):
        cc, b = t // B, t % B
        off = b * S_PER_W + cc * CHUNK
        return pltpu.async_copy(
            table_hbm.at[idx_v.at[pl.ds(off, CHUNK)]],
            rows[t % 3], gsem[t % 3])

    def start_pe(cc):
        return pltpu.async_copy(
            pe_hbm.at[pl.ds(w0 + cc * CHUNK, CHUNK)],
            pes[cc % 2], psem[cc % 2])

    g_h = [None, None, None]
    p_h = [None, None]
    s_h = [None, None, None]
    g_h[0] = start_gather(0)
    g_h[1] = start_gather(1)
    p_h[0] = start_pe(0)

    for t in range(N_CHUNKS):
        cc, b = t // B, t % B
        rb = t % 3
        if b == 0:
            p_h[cc % 2].wait()
        g_h[rb].wait()

        def row_body(r, _, _rb=rb, _pb=cc % 2):
            for d in range(D_VECS):
                sl = pl.ds(d * LANES, LANES)
                rows[_rb][r, sl] = rows[_rb][r, sl] * SCALE + pes[_pb][r, sl]
            return 0

        lax.fori_loop(0, CHUNK, row_body, 0)

        s_h[rb] = pltpu.async_copy(
            rows[rb], out_hbm.at[pl.ds(flat_base(t), CHUNK)], ssem[rb])

        nxt = t + 2
        if nxt < N_CHUNKS:
            if t >= 1:
                s_h[nxt % 3].wait()
            g_h[nxt % 3] = start_gather(nxt)
        if b == 0 and cc + 1 < N_PCH:
            p_h[(cc + 1) % 2] = start_pe(cc + 1)

    for t in range(N_CHUNKS - 3, N_CHUNKS):
        s_h[t % 3].wait()


@jax.jit
def _embed(x, pe, table):
    mesh = plsc.VectorSubcoreMesh(core_axis_name="c", subcore_axis_name="s")
    k = functools.partial(
        pl.kernel,
        mesh=mesh,
        out_type=jax.ShapeDtypeStruct((N_ROWS, D), jnp.float32),
        scratch_types=[
            pltpu.VMEM((B * S_PER_W,), jnp.int32),
            pltpu.VMEM((CHUNK, D), jnp.float32),
            pltpu.VMEM((CHUNK, D), jnp.float32),
            pltpu.VMEM((CHUNK, D), jnp.float32),
            pltpu.VMEM((CHUNK, D), jnp.float32),
            pltpu.VMEM((CHUNK, D), jnp.float32),
            pltpu.SemaphoreType.DMA,
            pltpu.SemaphoreType.DMA,
            pltpu.SemaphoreType.DMA,
            pltpu.SemaphoreType.DMA,
            pltpu.SemaphoreType.DMA,
            pltpu.SemaphoreType.DMA,
            pltpu.SemaphoreType.DMA,
            pltpu.SemaphoreType.DMA,
        ],
    )(_sc_body)
    return k(x, pe, table)


def kernel(x, token_table, pe):
    out = _embed(x.astype(jnp.int32), pe, token_table)
    return out.reshape(B, S, D)


# speedup vs baseline: 1.7057x; 1.0253x over previous
import functools

import jax
import jax.numpy as jnp
from jax import lax
from jax.experimental import pallas as pl
from jax.experimental.pallas import tpu as pltpu
from jax.experimental.pallas import tpu_sc as plsc

B = 4
S = 4096
D = 768
N_ROWS = B * S
NC = 2
NS = 16
NW = NC * NS
S_PER_W = S // NW
CHUNK = 32
N_PCH = S_PER_W // CHUNK
N_CHUNKS = N_PCH * B
LANES = 16
D_VECS = D // LANES
SCALE = 27.712812921102035


def _sc_body(x_hbm, pe_hbm, table_hbm, out_hbm,
             idx_v, r0, r1, r2, pv0, pv1,
             g0, g1, g2, p0, p1, s0_, s1_, s2_):
    rows = [r0, r1, r2]
    pes = [pv0, pv1]
    gsem = [g0, g1, g2]
    psem = [p0, p1]
    ssem = [s0_, s1_, s2_]

    wid = lax.axis_index("s") * NC + lax.axis_index("c")
    w0 = wid * S_PER_W

    idx_h = [
        pltpu.make_async_copy(x_hbm.at[b, pl.ds(w0, S_PER_W)],
                              idx_v.at[pl.ds(b * S_PER_W, S_PER_W)], p1)
        for b in range(B)
    ]
    for h in idx_h:
        h.start()
    for h in idx_h:
        h.wait()

    def flat_base(t):
        cc, b = t // B, t % B
        return b * S + w0 + cc * CHUNK

    def start_gather(t):
        cc, b = t // B, t % B
        off = b * S_PER_W + cc * CHUNK
        return pltpu.async_copy(
            table_hbm.at[idx_v.at[pl.ds(off, CHUNK)]],
            rows[t % 3], gsem[t % 3])

    def start_pe(cc):
        return pltpu.async_copy(
            pe_hbm.at[pl.ds(w0 + cc * CHUNK, CHUNK)],
            pes[cc % 2], psem[cc % 2])

    g_h = [None, None, None]
    p_h = [None, None]
    s_h = [None, None, None]
    g_h[0] = start_gather(0)
    g_h[1] = start_gather(1)
    g_h[2] = start_gather(2)
    p_h[0] = start_pe(0)

    for t in range(N_CHUNKS):
        cc, b = t // B, t % B
        rb = t % 3
        if b == 0:
            p_h[cc % 2].wait()
        g_h[rb].wait()

        def row_body(r, _, _rb=rb, _pb=cc % 2):
            for d in range(D_VECS):
                sl = pl.ds(d * LANES, LANES)
                rows[_rb][r, sl] = rows[_rb][r, sl] * SCALE + pes[_pb][r, sl]
            return 0

        lax.fori_loop(0, CHUNK, row_body, 0)

        s_h[rb] = pltpu.async_copy(
            rows[rb], out_hbm.at[pl.ds(flat_base(t), CHUNK)], ssem[rb])

        nxt = t + 2
        if nxt < N_CHUNKS and t >= 1:
            s_h[nxt % 3].wait()
            g_h[nxt % 3] = start_gather(nxt)
        if b == 0 and cc + 1 < N_PCH:
            p_h[(cc + 1) % 2] = start_pe(cc + 1)

    for t in range(N_CHUNKS - 3, N_CHUNKS):
        s_h[t % 3].wait()


@jax.jit
def _embed(x, pe, table):
    mesh = plsc.VectorSubcoreMesh(core_axis_name="c", subcore_axis_name="s")
    k = functools.partial(
        pl.kernel,
        mesh=mesh,
        out_type=jax.ShapeDtypeStruct((N_ROWS, D), jnp.float32),
        scratch_types=[
            pltpu.VMEM((B * S_PER_W,), jnp.int32),
            pltpu.VMEM((CHUNK, D), jnp.float32),
            pltpu.VMEM((CHUNK, D), jnp.float32),
            pltpu.VMEM((CHUNK, D), jnp.float32),
            pltpu.VMEM((CHUNK, D), jnp.float32),
            pltpu.VMEM((CHUNK, D), jnp.float32),
            pltpu.SemaphoreType.DMA,
            pltpu.SemaphoreType.DMA,
            pltpu.SemaphoreType.DMA,
            pltpu.SemaphoreType.DMA,
            pltpu.SemaphoreType.DMA,
            pltpu.SemaphoreType.DMA,
            pltpu.SemaphoreType.DMA,
            pltpu.SemaphoreType.DMA,
        ],
    )(_sc_body)
    return k(x, pe, table)


def kernel(x, token_table, pe):
    out = _embed(x.astype(jnp.int32), pe, token_table)
    return out.reshape(B, S, D)
